# Initial kernel scaffold; baseline (speedup 1.0000x reference)
import jax
import jax.numpy as jnp
from jax.experimental import pallas as pl


def kernel(feature, pm25_hist, adj_mat, angles, conv_W0, conv_W1, conv_b, gru_Wih, gru_Whh, gru_bih, gru_bhh, fc_W, fc_b):
    raise NotImplementedError("write your pallas kernel here")


# speedup vs baseline: 346.7061x; 346.7061x over previous
import functools
import math

import jax
import jax.numpy as jnp
from jax.experimental import pallas as pl
from jax.experimental.pallas import tpu as pltpu

_FORE = 12
_BC = 4


def _dgc_gru_kernel(
    feat_ref,
    pm_ref,
    adj_ref,
    ang_ref,
    w0c_ref,
    w1c_ref,
    wirc_ref,
    wizc_ref,
    winc_ref,
    ur_ref,
    uz_ref,
    un_ref,
    wx_ref,
    wg_ref,
    bih_ref,
    bhh_ref,
    fcw_ref,
    scal_ref,
    out_ref,
    hn_ref,
    xn_ref,
    c1_ref,
    c2_ref,
    msk_ref,
    *, n_nodes, chunk,
):
    i = pl.program_id(0)
    bc = pl.program_id(1)

    @pl.when(jnp.logical_and(i == 0, bc == 0))
    def _init_static():
        ang = ang_ref[...]
        c1_ref[...] = jnp.cos(ang - (math.pi / 2.0))
        c2_ref[...] = jnp.cos(ang)
        msk_ref[...] = (adj_ref[...] != 0).astype(jnp.float32)

    @pl.when(i == 0)
    def _init_state():
        hn_ref[pl.ds(bc * chunk, chunk), :] = jnp.zeros(
            (chunk, hn_ref.shape[1]), jnp.float32)
        xn_ref[pl.ds(bc, 1), :] = pm_ref[0]

    cf = feat_ref[0]
    xn_c = xn_ref[pl.ds(bc, 1), :].reshape(chunk)
    hn_c = hn_ref[pl.ds(bc * chunk, chunk), :]

    w0x = scal_ref[0, 0]
    w1x = scal_ref[0, 1]
    conv_b = scal_ref[0, 2]
    fc_b = scal_ref[0, 3]

    u10 = cf[0:n_nodes, 0]
    v10 = cf[0:n_nodes, 1]
    gate = (v10[:, None] * c1_ref[...] + u10[:, None] * c2_ref[...]) >= 0.5
    w = gate.astype(jnp.float32) * msk_ref[...]
    deg = jnp.sum(w, axis=1)
    deg_safe = jnp.where(deg > 0, deg, 1.0)
    dis = jnp.where(deg > 0, 1.0 / jnp.sqrt(deg_safe), 0.0)
    xn0 = xn_ref[0, 0:n_nodes]
    cf0 = cf[0:n_nodes, :]
    y0 = xn0 * w1x + jnp.dot(cf0, w1c_ref[...],
                             preferred_element_type=jnp.float32)[:, 0]
    t = dis * y0
    g = -(dis * jnp.dot(t, w, preferred_element_type=jnp.float32))
    ge = jnp.concatenate([g, jnp.zeros((chunk - n_nodes,), jnp.float32)])
    gm = ge * jnp.where(bc == 0, 1.0, 0.0)

    gcn = (xn_c * w0x
           + jnp.dot(cf, w0c_ref[...], preferred_element_type=jnp.float32)[:, 0]
           + gm + conv_b)
    x_gcn = jax.nn.sigmoid(gcn)

    def gi(wc_ref, k):
        return (jnp.dot(cf, wc_ref[...], preferred_element_type=jnp.float32)
                + xn_c[:, None] * wx_ref[k]
                + x_gcn[:, None] * wg_ref[k]
                + bih_ref[k])

    def gh(u_ref, k):
        return (jnp.dot(hn_c, u_ref[...], preferred_element_type=jnp.float32)
                + bhh_ref[k])

    r = jax.nn.sigmoid(gi(wirc_ref, 0) + gh(ur_ref, 0))
    z = jax.nn.sigmoid(gi(wizc_ref, 1) + gh(uz_ref, 1))
    nn = jnp.tanh(gi(winc_ref, 2) + r * gh(un_ref, 2))
    hn_new = (1.0 - z) * nn + z * hn_c

    xn_new = jnp.sum(hn_new * fcw_ref[0][None, :], axis=1) + fc_b

    hn_ref[pl.ds(bc * chunk, chunk), :] = hn_new
    xn_ref[pl.ds(bc, 1), :] = xn_new.reshape(1, chunk)
    out_ref[0, 0, 0, :] = xn_new


def kernel(feature, pm25_hist, adj_mat, angles, conv_W0, conv_W1, conv_b,
           gru_Wih, gru_Whh, gru_bih, gru_bhh, fc_W, fc_b):
    B, T, N, D = feature.shape
    fore = _FORE
    hist = T - fore
    H = gru_Whh.shape[1]
    NT = B * N
    chunk = NT // _BC

    feat = feature[:, hist:].transpose(1, 0, 2, 3).reshape(fore, NT, D)
    pm_last = pm25_hist[:, -1, :, 0].reshape(_BC, 1, chunk)

    w0c = conv_W0[1:]
    w1c = conv_W1[1:]
    wirc = gru_Wih[0:H, 1:1 + D].T
    wizc = gru_Wih[H:2 * H, 1:1 + D].T
    winc = gru_Wih[2 * H:3 * H, 1:1 + D].T
    ur = gru_Whh[0:H].T
    uz = gru_Whh[H:2 * H].T
    un = gru_Whh[2 * H:3 * H].T
    wx = gru_Wih[:, 0].reshape(3, H)
    wg = gru_Wih[:, 1 + D].reshape(3, H)
    bih = gru_bih.reshape(3, H)
    bhh = gru_bhh.reshape(3, H)
    fcw = fc_W.reshape(1, H)
    scal = jnp.stack([conv_W0[0, 0], conv_W1[0, 0],
                      conv_b[0], fc_b[0]]).reshape(1, 4)

    grid = (fore, _BC)
    full = lambda shape: pl.BlockSpec(shape, lambda i, bc: (0,) * len(shape))

    out = pl.pallas_call(
        functools.partial(_dgc_gru_kernel, n_nodes=N, chunk=chunk),
        grid=grid,
        in_specs=[
            pl.BlockSpec((1, chunk, D), lambda i, bc: (i, bc, 0)),
            pl.BlockSpec((1, 1, chunk), lambda i, bc: (bc, 0, 0)),
            full((N, N)),
            full((N, N)),
            full((D, 1)),
            full((D, 1)),
            full((D, H)),
            full((D, H)),
            full((D, H)),
            full((H, H)),
            full((H, H)),
            full((H, H)),
            full((3, H)),
            full((3, H)),
            full((3, H)),
            full((3, H)),
            full((1, H)),
            full((1, 4)),
        ],
        out_specs=pl.BlockSpec((1, 1, 1, chunk), lambda i, bc: (i, bc, 0, 0)),
        out_shape=jax.ShapeDtypeStruct((fore, _BC, 1, chunk), jnp.float32),
        scratch_shapes=[
            pltpu.VMEM((NT, H), jnp.float32),
            pltpu.VMEM((_BC, chunk), jnp.float32),
            pltpu.VMEM((N, N), jnp.float32),
            pltpu.VMEM((N, N), jnp.float32),
            pltpu.VMEM((N, N), jnp.float32),
        ],
    )(feat, pm_last, adj_mat, angles, w0c, w1c, wirc, wizc, winc,
      ur, uz, un, wx, wg, bih, bhh, fcw, scal)

    return out.reshape(fore, B, N).transpose(1, 0, 2)[..., None]


# speedup vs baseline: 948.9484x; 2.7370x over previous
import functools
import math

import jax
import jax.numpy as jnp
from jax.experimental import pallas as pl
from jax.experimental.pallas import tpu as pltpu

_FORE = 12
_BC = 2


def _dgc_gru_kernel(
    feat_ref,
    wind_ref,
    pm_ref,
    adj_ref,
    ang_ref,
    w0c_ref,
    w1c_ref,
    wic_ref,
    u_ref,
    wx_ref,
    wg_ref,
    bih_ref,
    bhh_ref,
    fcw_ref,
    scal_ref,
    out_ref,
    hn_ref,
    xn_ref,
    c1_ref,
    c2_ref,
    msk_ref,
    *, n_nodes, chunk, hid,
):
    i = pl.program_id(0)
    bc = pl.program_id(1)

    @pl.when(jnp.logical_and(i == 0, bc == 0))
    def _init_static():
        ang = ang_ref[...]
        c1_ref[...] = jnp.cos(ang - (math.pi / 2.0))
        c2_ref[...] = jnp.cos(ang)
        msk_ref[...] = (adj_ref[...] != 0).astype(jnp.float32)

    @pl.when(i == 0)
    def _init_state():
        hn_ref[:, pl.ds(bc * chunk, chunk)] = jnp.zeros(
            (hid, chunk), jnp.float32)
        xn_ref[pl.ds(bc, 1), :] = pm_ref[0]

    cf_t = feat_ref[0]
    xn_c = xn_ref[pl.ds(bc, 1), :].reshape(chunk)
    hn_c = hn_ref[:, pl.ds(bc * chunk, chunk)]

    w0x = scal_ref[0, 0]
    w1x = scal_ref[0, 1]
    conv_b = scal_ref[0, 2]
    fc_b = scal_ref[0, 3]

    u_col = wind_ref[0, :, 0:1]
    v_col = wind_ref[0, :, 1:2]
    gate = (v_col * c1_ref[...] + u_col * c2_ref[...]) >= 0.5
    w = gate.astype(jnp.float32) * msk_ref[...]
    deg = jnp.sum(w, axis=1)
    deg_safe = jnp.where(deg > 0, deg, 1.0)
    dis = jnp.where(deg > 0, 1.0 / jnp.sqrt(deg_safe), 0.0)
    xn0 = xn_ref[0, 0:n_nodes]
    y0 = (xn0 * w1x
          + jnp.dot(w1c_ref[...], cf_t[:, 0:n_nodes],
                    preferred_element_type=jnp.float32)[0])
    t = dis * y0
    g = -(dis * jnp.dot(t, w, preferred_element_type=jnp.float32))
    ge = jnp.concatenate([g, jnp.zeros((chunk - n_nodes,), jnp.float32)])
    gm = ge * jnp.where(bc == 0, 1.0, 0.0)

    gcn = (xn_c * w0x
           + jnp.dot(w0c_ref[...], cf_t,
                     preferred_element_type=jnp.float32)[0]
           + gm + conv_b)
    x_gcn = jax.nn.sigmoid(gcn)

    gi = (jnp.dot(wic_ref[...], cf_t, preferred_element_type=jnp.float32)
          + xn_c[None, :] * wx_ref[...]
          + x_gcn[None, :] * wg_ref[...]
          + bih_ref[...])
    gh = (jnp.dot(u_ref[...], hn_c, preferred_element_type=jnp.float32)
          + bhh_ref[...])

    r = jax.nn.sigmoid(gi[0:hid] + gh[0:hid])
    z = jax.nn.sigmoid(gi[hid:2 * hid] + gh[hid:2 * hid])
    nn = jnp.tanh(gi[2 * hid:3 * hid] + r * gh[2 * hid:3 * hid])
    hn_new = (1.0 - z) * nn + z * hn_c

    xn_new = jnp.sum(hn_new * fcw_ref[...], axis=0) + fc_b

    hn_ref[:, pl.ds(bc * chunk, chunk)] = hn_new
    xn_ref[pl.ds(bc, 1), :] = xn_new.reshape(1, chunk)
    out_ref[0, 0, 0, :] = xn_new


def kernel(feature, pm25_hist, adj_mat, angles, conv_W0, conv_W1, conv_b,
           gru_Wih, gru_Whh, gru_bih, gru_bhh, fc_W, fc_b):
    B, T, N, D = feature.shape
    fore = _FORE
    hist = T - fore
    H = gru_Whh.shape[1]
    NT = B * N
    chunk = NT // _BC

    ftail = feature[:, hist:]
    feat = ftail.transpose(1, 3, 0, 2).reshape(fore, D, NT)
    wind = ftail[0, :, :, 0:2]
    pm_last = pm25_hist[:, -1, :, 0].reshape(_BC, 1, chunk)

    w0c = conv_W0[1:].T
    w1c = conv_W1[1:].T
    wic = gru_Wih[:, 1:1 + D]
    wx = gru_Wih[:, 0:1]
    wg = gru_Wih[:, 1 + D:2 + D]
    bih = gru_bih[:, None]
    bhh = gru_bhh[:, None]
    fcw = fc_W.reshape(H, 1)
    scal = jnp.stack([conv_W0[0, 0], conv_W1[0, 0],
                      conv_b[0], fc_b[0]]).reshape(1, 4)

    grid = (fore, _BC)
    full = lambda shape: pl.BlockSpec(shape, lambda i, bc: (0,) * len(shape))

    out = pl.pallas_call(
        functools.partial(_dgc_gru_kernel, n_nodes=N, chunk=chunk, hid=H),
        grid=grid,
        in_specs=[
            pl.BlockSpec((1, D, chunk), lambda i, bc: (i, 0, bc)),
            pl.BlockSpec((1, N, 2), lambda i, bc: (i, 0, 0)),
            pl.BlockSpec((1, 1, chunk), lambda i, bc: (bc, 0, 0)),
            full((N, N)),
            full((N, N)),
            full((1, D)),
            full((1, D)),
            full((3 * H, D)),
            full((3 * H, H)),
            full((3 * H, 1)),
            full((3 * H, 1)),
            full((3 * H, 1)),
            full((3 * H, 1)),
            full((H, 1)),
            full((1, 4)),
        ],
        out_specs=pl.BlockSpec((1, 1, 1, chunk), lambda i, bc: (i, bc, 0, 0)),
        out_shape=jax.ShapeDtypeStruct((fore, _BC, 1, chunk), jnp.float32),
        scratch_shapes=[
            pltpu.VMEM((H, NT), jnp.float32),
            pltpu.VMEM((_BC, chunk), jnp.float32),
            pltpu.VMEM((N, N), jnp.float32),
            pltpu.VMEM((N, N), jnp.float32),
            pltpu.VMEM((N, N), jnp.float32),
        ],
    )(feat, wind, pm_last, adj_mat, angles, w0c, w1c, wic,
      gru_Whh, wx, wg, bih, bhh, fcw, scal)

    return out.reshape(fore, B, N).transpose(1, 0, 2)[..., None]


# speedup vs baseline: 1074.2709x; 1.1321x over previous
import functools
import math

import jax
import jax.numpy as jnp
from jax.experimental import pallas as pl
from jax.experimental.pallas import tpu as pltpu

_FORE = 12
_BC = 2


def _dgc_gru_kernel(
    feat_ref,
    wind_ref,
    pm_ref,
    adj_ref,
    ang_ref,
    wgcn_ref,
    wy_ref,
    wgi_ref,
    wgh_ref,
    fcw_ref,
    scal_ref,
    out_ref,
    hn_ref,
    xn_ref,
    g_ref,
    c1_ref,
    c2_ref,
    msk_ref,
    *, n_nodes, chunk, hid,
):
    i = pl.program_id(0)
    bc = pl.program_id(1)

    @pl.when(jnp.logical_and(i == 0, bc == 0))
    def _init_static():
        ang = ang_ref[...]
        c1_ref[...] = jnp.cos(ang - (math.pi / 2.0))
        c2_ref[...] = jnp.cos(ang)
        msk_ref[...] = (adj_ref[...] != 0).astype(jnp.float32)

    @pl.when(i == 0)
    def _init_state():
        hn_ref[:, pl.ds(bc * chunk, chunk)] = jnp.concatenate(
            [jnp.zeros((hid, chunk), jnp.float32),
             jnp.ones((1, chunk), jnp.float32)], axis=0)
        xn_ref[pl.ds(bc, 1), :] = pm_ref[0]

    cf_t = feat_ref[0]
    xn_c = xn_ref[pl.ds(bc, 1), :]
    hn_aug = hn_ref[:, pl.ds(bc * chunk, chunk)]
    hn_c = hn_aug[0:hid]
    fc_b = scal_ref[0, 0]

    aug0 = jnp.concatenate(
        [cf_t, xn_c, jnp.ones((1, chunk), jnp.float32)], axis=0)

    @pl.when(bc == 0)
    def _graph():
        u_col = wind_ref[0, :, 0:1]
        v_col = wind_ref[0, :, 1:2]
        gate = (v_col * c1_ref[...] + u_col * c2_ref[...]) >= 0.5
        w = gate.astype(jnp.float32) * msk_ref[...]
        deg = jnp.sum(w, axis=1)
        deg_safe = jnp.where(deg > 0, deg, 1.0)
        dis = jnp.where(deg > 0, 1.0 / jnp.sqrt(deg_safe), 0.0)
        y0 = jnp.dot(wy_ref[...], aug0[:, 0:n_nodes],
                     preferred_element_type=jnp.float32)[0]
        t = dis * y0
        g_ref[...] = -(dis * jnp.dot(t, w, preferred_element_type=jnp.float32)
                       )[None, :]

    ge = jnp.concatenate(
        [g_ref[0], jnp.zeros((chunk - n_nodes,), jnp.float32)])
    gm = ge * jnp.where(bc == 0, 1.0, 0.0)

    gcn = jnp.dot(wgcn_ref[...], aug0,
                  preferred_element_type=jnp.float32)[0] + gm
    x_gcn = jax.nn.sigmoid(gcn)

    aug = jnp.concatenate([aug0, x_gcn[None, :]], axis=0)
    gi = jnp.dot(wgi_ref[...], aug,
                 preferred_element_type=jnp.float32)
    gh = jnp.dot(wgh_ref[...], hn_aug,
                 preferred_element_type=jnp.float32)

    rz = jax.nn.sigmoid(gi[0:2 * hid] + gh[0:2 * hid])
    r = rz[0:hid]
    z = rz[hid:2 * hid]
    nn = jnp.tanh(gi[2 * hid:3 * hid] + r * gh[2 * hid:3 * hid])
    hn_new = (1.0 - z) * nn + z * hn_c

    xn_new = jnp.dot(fcw_ref[...], hn_new,
                     preferred_element_type=jnp.float32)[0] + fc_b

    hn_ref[0:hid, pl.ds(bc * chunk, chunk)] = hn_new
    xn_ref[pl.ds(bc, 1), :] = xn_new[None, :]
    out_ref[0, 0, 0, :] = xn_new


def kernel(feature, pm25_hist, adj_mat, angles, conv_W0, conv_W1, conv_b,
           gru_Wih, gru_Whh, gru_bih, gru_bhh, fc_W, fc_b):
    B, T, N, D = feature.shape
    fore = _FORE
    hist = T - fore
    H = gru_Whh.shape[1]
    NT = B * N
    chunk = NT // _BC

    ftail = feature[:, hist:]
    feat = ftail.transpose(1, 3, 0, 2).reshape(fore, D, NT)
    wind = ftail[0, :, :, 0:2]
    pm_last = pm25_hist[:, -1, :, 0].reshape(_BC, 1, chunk)

    zero1 = jnp.zeros((1, 1), jnp.float32)
    wgcn = jnp.concatenate(
        [conv_W0[1:].T, conv_W0[0:1].T, conv_b[None, :]], axis=1)
    wy = jnp.concatenate(
        [conv_W1[1:].T, conv_W1[0:1].T, zero1], axis=1)
    wgi = jnp.concatenate(
        [gru_Wih[:, 1:1 + D], gru_Wih[:, 0:1], gru_bih[:, None],
         gru_Wih[:, 1 + D:2 + D]], axis=1)
    wgh = jnp.concatenate([gru_Whh, gru_bhh[:, None]], axis=1)
    fcw = fc_W.reshape(1, H)
    scal = fc_b.reshape(1, 1)

    grid = (fore, _BC)
    full = lambda shape: pl.BlockSpec(shape, lambda i, bc: (0,) * len(shape))

    out = pl.pallas_call(
        functools.partial(_dgc_gru_kernel, n_nodes=N, chunk=chunk, hid=H),
        grid=grid,
        in_specs=[
            pl.BlockSpec((1, D, chunk), lambda i, bc: (i, 0, bc)),
            pl.BlockSpec((1, N, 2), lambda i, bc: (i, 0, 0)),
            pl.BlockSpec((1, 1, chunk), lambda i, bc: (bc, 0, 0)),
            full((N, N)),
            full((N, N)),
            full((1, D + 2)),
            full((1, D + 2)),
            full((3 * H, D + 3)),
            full((3 * H, H + 1)),
            full((1, H)),
            full((1, 1)),
        ],
        out_specs=pl.BlockSpec((1, 1, 1, chunk), lambda i, bc: (i, bc, 0, 0)),
        out_shape=jax.ShapeDtypeStruct((fore, _BC, 1, chunk), jnp.float32),
        scratch_shapes=[
            pltpu.VMEM((H + 1, NT), jnp.float32),
            pltpu.VMEM((_BC, chunk), jnp.float32),
            pltpu.VMEM((1, N), jnp.float32),
            pltpu.VMEM((N, N), jnp.float32),
            pltpu.VMEM((N, N), jnp.float32),
            pltpu.VMEM((N, N), jnp.float32),
        ],
    )(feat, wind, pm_last, adj_mat, angles, wgcn, wy, wgi, wgh, fcw, scal)

    return out.reshape(fore, B, N).transpose(1, 0, 2)[..., None]


# speedup vs baseline: 1148.7084x; 1.0693x over previous
import functools
import math

import jax
import jax.numpy as jnp
from jax.experimental import pallas as pl
from jax.experimental.pallas import tpu as pltpu

_FORE = 12


def _dgc_gru_kernel(
    feat_ref,
    wind_ref,
    pm_ref,
    adj_ref,
    ang_ref,
    wgcn_ref,
    wy_ref,
    wgi_ref,
    wgh_ref,
    fcw_ref,
    scal_ref,
    out_ref,
    hn_ref,
    xn_ref,
    c1_ref,
    c2_ref,
    *, n_nodes, nt, hid,
):
    i = pl.program_id(0)

    @pl.when(i == 0)
    def _init():
        ang = ang_ref[...]
        msk = (adj_ref[...] != 0).astype(jnp.float32)
        c1_ref[...] = jnp.cos(ang - (math.pi / 2.0)) * msk
        c2_ref[...] = jnp.cos(ang) * msk
        hn_ref[...] = jnp.concatenate(
            [jnp.zeros((hid, nt), jnp.float32),
             jnp.ones((1, nt), jnp.float32)], axis=0)
        xn_ref[...] = pm_ref[0]

    cf_t = feat_ref[0]
    xn_c = xn_ref[...]
    hn_aug = hn_ref[...]
    hn_c = hn_aug[0:hid]
    fc_b = scal_ref[0, 0]

    aug0 = jnp.concatenate(
        [cf_t, xn_c, jnp.ones((1, nt), jnp.float32)], axis=0)

    u_col = wind_ref[0, :, 0:1]
    v_col = wind_ref[0, :, 1:2]
    w = ((v_col * c1_ref[...] + u_col * c2_ref[...]) >= 0.5
         ).astype(jnp.float32)
    deg = jnp.sum(w, axis=1)
    deg_safe = jnp.where(deg > 0, deg, 1.0)
    dis = jnp.where(deg > 0, 1.0 / jnp.sqrt(deg_safe), 0.0)
    y0 = jnp.dot(wy_ref[...], aug0[:, 0:n_nodes],
                 preferred_element_type=jnp.float32)[0]
    t = dis * y0
    g = -(dis * jnp.dot(t, w, preferred_element_type=jnp.float32))
    ge = jnp.concatenate([g, jnp.zeros((nt - n_nodes,), jnp.float32)])

    gcn = jnp.dot(wgcn_ref[...], aug0,
                  preferred_element_type=jnp.float32)[0] + ge
    x_gcn = jax.nn.sigmoid(gcn)

    aug = jnp.concatenate([aug0, x_gcn[None, :]], axis=0)
    gi = jnp.dot(wgi_ref[...], aug,
                 preferred_element_type=jnp.float32)
    gh = jnp.dot(wgh_ref[...], hn_aug,
                 preferred_element_type=jnp.float32)

    rz = jax.nn.sigmoid(gi[0:2 * hid] + gh[0:2 * hid])
    r = rz[0:hid]
    z = rz[hid:2 * hid]
    nn = jnp.tanh(gi[2 * hid:3 * hid] + r * gh[2 * hid:3 * hid])
    hn_new = nn + z * (hn_c - nn)

    xn_new = jnp.dot(fcw_ref[...], hn_new,
                     preferred_element_type=jnp.float32)[0] + fc_b

    hn_ref[0:hid, :] = hn_new
    xn_ref[...] = xn_new[None, :]
    out_ref[0, 0, :] = xn_new


def kernel(feature, pm25_hist, adj_mat, angles, conv_W0, conv_W1, conv_b,
           gru_Wih, gru_Whh, gru_bih, gru_bhh, fc_W, fc_b):
    B, T, N, D = feature.shape
    fore = _FORE
    hist = T - fore
    H = gru_Whh.shape[1]
    NT = B * N

    ftail = feature[:, hist:]
    feat = ftail.transpose(1, 3, 0, 2).reshape(fore, D, NT)
    wind = ftail[0, :, :, 0:2]
    pm_last = pm25_hist[:, -1, :, 0].reshape(1, 1, NT)

    zero1 = jnp.zeros((1, 1), jnp.float32)
    wgcn = jnp.concatenate(
        [conv_W0[1:].T, conv_W0[0:1].T, conv_b[None, :]], axis=1)
    wy = jnp.concatenate(
        [conv_W1[1:].T, conv_W1[0:1].T, zero1], axis=1)
    wgi = jnp.concatenate(
        [gru_Wih[:, 1:1 + D], gru_Wih[:, 0:1], gru_bih[:, None],
         gru_Wih[:, 1 + D:2 + D]], axis=1)
    wgh = jnp.concatenate([gru_Whh, gru_bhh[:, None]], axis=1)
    fcw = fc_W.reshape(1, H)
    scal = fc_b.reshape(1, 1)

    grid = (fore,)
    full = lambda shape: pl.BlockSpec(shape, lambda i: (0,) * len(shape))

    out = pl.pallas_call(
        functools.partial(_dgc_gru_kernel, n_nodes=N, nt=NT, hid=H),
        grid=grid,
        in_specs=[
            pl.BlockSpec((1, D, NT), lambda i: (i, 0, 0)),
            pl.BlockSpec((1, N, 2), lambda i: (i, 0, 0)),
            pl.BlockSpec((1, 1, NT), lambda i: (0, 0, 0)),
            full((N, N)),
            full((N, N)),
            full((1, D + 2)),
            full((1, D + 2)),
            full((3 * H, D + 3)),
            full((3 * H, H + 1)),
            full((1, H)),
            full((1, 1)),
        ],
        out_specs=pl.BlockSpec((1, 1, NT), lambda i: (i, 0, 0)),
        out_shape=jax.ShapeDtypeStruct((fore, 1, NT), jnp.float32),
        scratch_shapes=[
            pltpu.VMEM((H + 1, NT), jnp.float32),
            pltpu.VMEM((1, NT), jnp.float32),
            pltpu.VMEM((N, N), jnp.float32),
            pltpu.VMEM((N, N), jnp.float32),
        ],
    )(feat, wind, pm_last, adj_mat, angles, wgcn, wy, wgi, wgh, fcw, scal)

    return out.reshape(fore, B, N).transpose(1, 0, 2)[..., None]
